# baseline (speedup 1.0000x reference)
import functools

import jax
import jax.numpy as jnp
from jax import lax
from jax.experimental import pallas as pl
from jax.experimental.pallas import tpu as pltpu
from jax.experimental.pallas import tpu_sc as plsc

_B = 16384
_F = 16
_H = 64
_MAXR = 5.0
_MINR = 0.5

_info = plsc.get_sparse_core_info()
_NC = _info.num_cores
_NS = _info.num_subcores
_NW = _NC * _NS
_CHUNK = 128
_B_PER_W = _B // _NW
_NCHUNK = _B_PER_W // _CHUNK

_mesh = plsc.VectorSubcoreMesh(core_axis_name="c", subcore_axis_name="s")


@functools.partial(
    pl.kernel,
    mesh=_mesh,
    out_type=[
        jax.ShapeDtypeStruct((_B, _F), jnp.float32),
        jax.ShapeDtypeStruct((_B, _F), jnp.float32),
    ],
    scratch_types=[
        pltpu.VMEM((_NCHUNK, _CHUNK), jnp.int32),
        pltpu.VMEM((_NCHUNK, _CHUNK), jnp.int32),
        pltpu.VMEM((_B_PER_W, _F), jnp.float32),
        pltpu.VMEM((_B_PER_W, _F), jnp.float32),
        pltpu.SemaphoreType.DMA,
        pltpu.SemaphoreType.DMA,
    ],
    compiler_params=pltpu.CompilerParams(use_tc_tiling_on_sc=False),
)
def _gather2(user_h, movie_h, U_h, M_h, uout_h, mout_h,
             uidx, midx, urows, mrows, usem, msem):
    wid = lax.axis_index("s") * _NC + lax.axis_index("c")
    base = wid * _B_PER_W
    pltpu.sync_copy(user_h.at[pl.ds(wid * _NCHUNK, _NCHUNK)], uidx)
    pltpu.sync_copy(movie_h.at[pl.ds(wid * _NCHUNK, _NCHUNK)], midx)
    copies = []
    for j in range(_NCHUNK):
        sl = pl.ds(j * _CHUNK, _CHUNK)
        copies.append(pltpu.async_copy(U_h.at[uidx.at[j]], urows.at[sl], usem))
        copies.append(pltpu.async_copy(M_h.at[midx.at[j]], mrows.at[sl], msem))
    for c in copies:
        c.wait()
    pltpu.sync_copy(urows, uout_h.at[pl.ds(base, _B_PER_W)])
    pltpu.sync_copy(mrows, mout_h.at[pl.ds(base, _B_PER_W)])


_ROWS = 2048


def _mlp_body(u_ref, m_ref, w1u_ref, w1m_ref, b1_ref, w2_ref, b2_ref, o_ref):
    h = (
        jnp.dot(u_ref[...], w1u_ref[...], preferred_element_type=jnp.float32)
        + jnp.dot(m_ref[...], w1m_ref[...], preferred_element_type=jnp.float32)
        + b1_ref[...]
    )
    h = jnp.maximum(h, 0.0)
    y = jnp.sum(h * w2_ref[...], axis=1, keepdims=True) + b2_ref[...]
    sig = 1.0 / (1.0 + jnp.exp(-y))
    o_ref[...] = sig * (_MAXR - _MINR) + _MINR


_mlp = pl.pallas_call(
    _mlp_body,
    grid=(_B // _ROWS,),
    in_specs=[
        pl.BlockSpec((_ROWS, _F), lambda i: (i, 0)),
        pl.BlockSpec((_ROWS, _F), lambda i: (i, 0)),
        pl.BlockSpec((_F, _H), lambda i: (0, 0)),
        pl.BlockSpec((_F, _H), lambda i: (0, 0)),
        pl.BlockSpec((1, _H), lambda i: (0, 0)),
        pl.BlockSpec((1, _H), lambda i: (0, 0)),
        pl.BlockSpec((1, 1), lambda i: (0, 0)),
    ],
    out_specs=pl.BlockSpec((_ROWS, 1), lambda i: (i, 0)),
    out_shape=jax.ShapeDtypeStruct((_B, 1), jnp.float32),
)


def kernel(user, movie, U, M, W1, b1, W2, b2):
    user2 = user.reshape(_NW * _NCHUNK, _CHUNK)
    movie2 = movie.reshape(_NW * _NCHUNK, _CHUNK)
    u_emb, m_emb = _gather2(user2, movie2, U, M)
    return _mlp(
        u_emb,
        m_emb,
        W1[:_F],
        W1[_F:],
        b1.reshape(1, _H),
        W2.reshape(1, _H),
        b2.reshape(1, 1),
    )
